# baseline (speedup 1.0000x reference)
import functools

import jax
import jax.numpy as jnp
from jax import lax
from jax.experimental import pallas as pl
from jax.experimental.pallas import tpu as pltpu
from jax.experimental.pallas import tpu_sc as plsc

NUM_EXPERTS = 8
TOP_K = 2
_NEG = -1e30
_TM = 384
_FB = 512


def _pack_bf16(y):
    dh = y.shape[1] // 2

    def rb(v):
        uv = jax.lax.bitcast_convert_type(v, jnp.uint32)
        return (uv + jnp.uint32(0x7FFF) + ((uv >> 16) & jnp.uint32(1))) >> 16

    packed = rb(y[:, :dh]) | (rb(y[:, dh:]) << 16)
    return jax.lax.bitcast_convert_type(packed, jnp.int32)


def _router_body(x_ref, gwp_ref, logits_ref, meta_ref, xp_ref):
    x = x_ref[...]
    xp_ref[...] = _pack_bf16(x)
    logits = jax.lax.dot_general(
        x, gwp_ref[...], (((1,), (1,)), ((), ())),
        preferred_element_type=jnp.float32)
    col = jax.lax.broadcasted_iota(jnp.int32, logits.shape, 1)
    valid = col < NUM_EXPERTS
    l = jnp.where(valid, logits, _NEG)
    m1 = jnp.max(l, axis=1, keepdims=True)
    idx1 = jnp.min(jnp.where(l == m1, col, 128), axis=1, keepdims=True)
    l2 = jnp.where(col == idx1, _NEG, l)
    m2 = jnp.max(l2, axis=1, keepdims=True)
    idx2 = jnp.min(jnp.where(l2 == m2, col, 128), axis=1, keepdims=True)
    b = jnp.exp(m2 - m1)
    w1 = 1.0 / (1.0 + b)
    w2 = b * w1
    logits_ref[...] = logits
    meta_ref[...] = jnp.where(col == 0, idx1.astype(jnp.float32),
                    jnp.where(col == 1, idx2.astype(jnp.float32),
                    jnp.where(col == 2, w1,
                    jnp.where(col == 3, w2, 0.0))))


def _router(x, gate_w):
    T, D = x.shape
    gwp = jnp.zeros((128, D), jnp.float32).at[:NUM_EXPERTS].set(gate_w)
    return pl.pallas_call(
        _router_body,
        out_shape=(jax.ShapeDtypeStruct((T, 128), jnp.float32),
                   jax.ShapeDtypeStruct((T, 128), jnp.float32),
                   jax.ShapeDtypeStruct((T, D // 2), jnp.int32)),
    )(x, gwp)


def _meta_body(ep_ref, pos_ref, steps_ref, *, tm):
    ep = ep_ref[...]
    R = ep.shape[0]
    i128 = jax.lax.broadcasted_iota(jnp.int32, (128, 128), 0)
    j128 = jax.lax.broadcasted_iota(jnp.int32, (128, 128), 1)
    ut = (i128 <= j128).astype(jnp.float32)
    iR = jax.lax.broadcasted_iota(jnp.int32, (R, R), 0)
    jR = jax.lax.broadcasted_iota(jnp.int32, (R, R), 1)
    slt = (jR < iR).astype(jnp.float32)

    pos = jnp.zeros(ep.shape, jnp.float32)
    tile_off = jnp.float32(0.0)
    tile_offs = []
    for e in range(NUM_EXPERTS):
        m = (ep == e).astype(jnp.float32)
        pin = jnp.dot(m, ut, preferred_element_type=jnp.float32)
        s_col = pin[:, 127:128]
        carry = jnp.dot(slt, jnp.broadcast_to(s_col, ep.shape),
                        preferred_element_type=jnp.float32)
        rank = pin - m + carry
        cnt = jnp.sum(m)
        pos = pos + m * (tile_off * tm + rank)
        tile_offs.append(tile_off)
        tile_off = tile_off + jnp.ceil(cnt / tm)
    total_tiles = tile_off

    ivec = jax.lax.broadcasted_iota(jnp.int32, (1, 128), 1).astype(jnp.float32)
    icl = jnp.minimum(ivec, total_tiles - 1.0)
    sexp = -jnp.ones((1, 128), jnp.float32)
    for e in range(NUM_EXPERTS):
        sexp = sexp + (icl >= tile_offs[e]).astype(jnp.float32)
    svalid = (ivec < total_tiles).astype(jnp.float32)

    pos_ref[...] = pos.astype(jnp.int32)
    r8 = jax.lax.broadcasted_iota(jnp.int32, (8, 128), 0)
    steps = jnp.where(r8 == 0, jnp.broadcast_to(sexp, (8, 128)),
            jnp.where(r8 == 1, jnp.broadcast_to(svalid, (8, 128)), 0.0))
    steps_ref[...] = steps.astype(jnp.int32)


def _metadata(ep, tm):
    R = ep.shape[0]
    return pl.pallas_call(
        functools.partial(_meta_body, tm=tm),
        out_shape=(jax.ShapeDtypeStruct((R, 128), jnp.int32),
                   jax.ShapeDtypeStruct((8, 128), jnp.int32)),
    )(ep)


def _sc_dispatch(x, tid3, pos3, s_pad):
    T, D = x.shape
    nw = tid3.shape[0]
    nchunk, cb = tid3.shape[1], tid3.shape[2]
    info = plsc.get_sparse_core_info()
    nc = info.num_cores

    @functools.partial(
        pl.kernel,
        mesh=plsc.VectorSubcoreMesh(core_axis_name="c", subcore_axis_name="s"),
        out_type=jax.ShapeDtypeStruct((s_pad, D), jnp.int32),
        scratch_types=[
            pltpu.VMEM((nchunk, cb), jnp.int32),
            pltpu.VMEM((nchunk, cb), jnp.int32),
            pltpu.VMEM((cb, D), jnp.int32),
            pltpu.VMEM((cb, D), jnp.int32),
            pltpu.SemaphoreType.DMA,
            pltpu.SemaphoreType.DMA,
            pltpu.SemaphoreType.DMA,
        ],
    )
    def dispatch(x_hbm, tid_hbm, pos_hbm, xs_hbm, idx_t, idx_p,
                 rows0, rows1, semg, sems0, sems1):
        wid = lax.axis_index("s") * nc + lax.axis_index("c")
        pltpu.sync_copy(tid_hbm.at[wid], idx_t)
        pltpu.sync_copy(pos_hbm.at[wid], idx_p)
        bufs = (rows0, rows1)
        ssems = (sems0, sems1)
        hg = {0: pltpu.async_copy(x_hbm.at[idx_t.at[0]], rows0, semg)}
        hs = {}
        for j in range(nchunk):
            buf = bufs[j % 2]
            hg[j].wait()
            hs[j] = pltpu.async_copy(buf, xs_hbm.at[idx_p.at[j]], ssems[j % 2])
            if j + 1 < nchunk:
                if j - 1 >= 0:
                    hs[j - 1].wait()
                hg[j + 1] = pltpu.async_copy(
                    x_hbm.at[idx_t.at[j + 1]], bufs[(j + 1) % 2], semg)
        if nchunk >= 2:
            hs[nchunk - 2].wait()
        hs[nchunk - 1].wait()

    return dispatch(x, tid3, pos3)


def _gmm_body(expert_s, valid_s, xs_ref, wg_ref, wu_ref, wd_ref, ys_ref):
    i = pl.program_id(0)

    @pl.when(valid_s[i] == 1)
    def _compute():
        xb = _unpack_bf16(xs_ref[...]).astype(jnp.bfloat16)
        wg = wg_ref[0].astype(jnp.bfloat16)
        wu = wu_ref[0].astype(jnp.bfloat16)
        wd = wd_ref[0].astype(jnp.bfloat16)
        g = jnp.dot(xb, wg, preferred_element_type=jnp.float32)
        u = jnp.dot(xb, wu, preferred_element_type=jnp.float32)
        h = (g * jax.lax.logistic(g) * u).astype(jnp.bfloat16)
        y = jnp.dot(h, wd, preferred_element_type=jnp.float32)
        dh = y.shape[1] // 2

        def rb(v):
            uv = jax.lax.bitcast_convert_type(v, jnp.uint32)
            return (uv + jnp.uint32(0x7FFF) + ((uv >> 16) & jnp.uint32(1))) >> 16

        packed = rb(y[:, :dh]) | (rb(y[:, dh:]) << 16)
        ys_ref[...] = jax.lax.bitcast_convert_type(packed, jnp.int32)


def _grouped_mlp(xs, w_gate, w_up, w_down, expert_arr, valid_arr, tm):
    s_pad, DP = xs.shape
    D = DP * 2
    E, _, F = w_gate.shape
    G = s_pad // tm
    grid_spec = pltpu.PrefetchScalarGridSpec(
        num_scalar_prefetch=2,
        grid=(G,),
        in_specs=[
            pl.BlockSpec((tm, DP), lambda i, es, vs: (i, 0)),
            pl.BlockSpec((1, D, F), lambda i, es, vs: (es[i], 0, 0)),
            pl.BlockSpec((1, D, F), lambda i, es, vs: (es[i], 0, 0)),
            pl.BlockSpec((1, F, D), lambda i, es, vs: (es[i], 0, 0)),
        ],
        out_specs=pl.BlockSpec((tm, D // 2), lambda i, es, vs: (i, 0)),
    )
    return pl.pallas_call(
        _gmm_body,
        grid_spec=grid_spec,
        out_shape=jax.ShapeDtypeStruct((s_pad, D // 2), jnp.int32),
        compiler_params=pltpu.CompilerParams(
            dimension_semantics=("arbitrary",),
            vmem_limit_bytes=100 * 1024 * 1024),
    )(expert_arr, valid_arr, xs, w_gate, w_up, w_down)


def _sc_collect(ys, p03, p13, T):
    s_pad, D = ys.shape
    nchunk, cb = p03.shape[1], p03.shape[2]
    per_w = nchunk * cb
    info = plsc.get_sparse_core_info()
    nc = info.num_cores

    @functools.partial(
        pl.kernel,
        mesh=plsc.VectorSubcoreMesh(core_axis_name="c", subcore_axis_name="s"),
        out_type=(jax.ShapeDtypeStruct((T, D), jnp.int32),
                  jax.ShapeDtypeStruct((T, D), jnp.int32)),
        scratch_types=[
            pltpu.VMEM((nchunk, cb), jnp.int32),
            pltpu.VMEM((nchunk, cb), jnp.int32),
            pltpu.VMEM((cb, D), jnp.int32),
            pltpu.VMEM((cb, D), jnp.int32),
            pltpu.SemaphoreType.DMA,
            pltpu.SemaphoreType.DMA,
            pltpu.SemaphoreType.DMA,
        ],
    )
    def collect(ys_hbm, p0_hbm, p1_hbm, y0_hbm, y1_hbm, idx0, idx1,
                buf0, buf1, semg, semw0, semw1):
        wid = lax.axis_index("s") * nc + lax.axis_index("c")
        base = wid * per_w
        pltpu.sync_copy(p0_hbm.at[wid], idx0)
        pltpu.sync_copy(p1_hbm.at[wid], idx1)
        steps = []
        for j in range(nchunk):
            steps.append((idx0, j, y0_hbm))
            steps.append((idx1, j, y1_hbm))
        ns = len(steps)
        bufs = (buf0, buf1)
        wsems = (semw0, semw1)
        ix0, j0, _ = steps[0]
        hg = {0: pltpu.async_copy(ys_hbm.at[ix0.at[j0]], buf0, semg)}
        hw = {}
        for s in range(ns):
            buf = bufs[s % 2]
            _, j, dst = steps[s]
            hg[s].wait()
            hw[s] = pltpu.async_copy(
                buf, dst.at[pl.ds(base + j * cb, cb)], wsems[s % 2])
            if s + 1 < ns:
                if s - 1 >= 0:
                    hw[s - 1].wait()
                ixn, jn, _ = steps[s + 1]
                hg[s + 1] = pltpu.async_copy(
                    ys_hbm.at[ixn.at[jn]], bufs[(s + 1) % 2], semg)
        if ns >= 2:
            hw[ns - 2].wait()
        hw[ns - 1].wait()

    return collect(ys, p03, p13)


def _unpack_bf16(p):
    u = jax.lax.bitcast_convert_type(p, jnp.uint32)
    lo = jax.lax.bitcast_convert_type(u << 16, jnp.float32)
    hi = jax.lax.bitcast_convert_type(u & jnp.uint32(0xFFFF0000), jnp.float32)
    return jnp.concatenate([lo, hi], axis=1)


def _combine_body(meta_ref, y0_ref, y1_ref, out_ref):
    out_ref[...] = (_unpack_bf16(y0_ref[...]) * meta_ref[:, 2:3]
                    + _unpack_bf16(y1_ref[...]) * meta_ref[:, 3:4])


def _combine(meta, y0, y1):
    T, DP = y0.shape
    TN = 256
    return pl.pallas_call(
        _combine_body,
        grid=(T // TN,),
        in_specs=[
            pl.BlockSpec((TN, 128), lambda t: (t, 0)),
            pl.BlockSpec((TN, DP), lambda t: (t, 0)),
            pl.BlockSpec((TN, DP), lambda t: (t, 0)),
        ],
        out_specs=pl.BlockSpec((TN, DP * 2), lambda t: (t, 0)),
        out_shape=jax.ShapeDtypeStruct((T, DP * 2), jnp.float32),
    )(meta, y0, y1)


def kernel(hidden_states, gate_w, w_gate, w_up, w_down):
    B, S, D = hidden_states.shape
    T = B * S
    E, _, F = w_gate.shape
    x = hidden_states.reshape(T, D)

    logits128, meta, xp = _router(x, gate_w)
    router_logits = logits128[:, :NUM_EXPERTS]

    idx1 = meta[:, 0].astype(jnp.int32)
    idx2 = meta[:, 1].astype(jnp.int32)
    ep = jnp.concatenate([idx1, idx2]).reshape(TOP_K * T // 128, 128)

    G = (TOP_K * T) // _TM + NUM_EXPERTS
    s_pad = G * _TM
    pos, steps = _metadata(ep, _TM)
    pos_flat = pos.reshape(TOP_K * T)
    expert_arr = steps[0, :G]
    valid_arr = steps[1, :G]

    nw = 32
    tid3 = jnp.tile(jnp.arange(T, dtype=jnp.int32), TOP_K).reshape(nw, -1, 16)
    pos3 = pos_flat.reshape(nw, -1, 16)
    xs = _sc_dispatch(xp, tid3, pos3, s_pad)

    ys = _grouped_mlp(xs, w_gate, w_up, w_down, expert_arr, valid_arr, _TM)

    p03 = pos_flat[:T].reshape(nw, -1, 16)
    p13 = pos_flat[T:].reshape(nw, -1, 16)
    y0, y1 = _sc_collect(ys, p03, p13, T)

    final = _combine(meta, y0, y1)
    return final.reshape(B, S, D), router_logits


# speedup vs baseline: 1.0271x; 1.0271x over previous
import functools

import jax
import jax.numpy as jnp
from jax import lax
from jax.experimental import pallas as pl
from jax.experimental.pallas import tpu as pltpu
from jax.experimental.pallas import tpu_sc as plsc

NUM_EXPERTS = 8
TOP_K = 2
_NEG = -1e30
_TM = 512
_FB = 512


def _pack_bf16(y):
    dh = y.shape[1] // 2

    def rb(v):
        uv = jax.lax.bitcast_convert_type(v, jnp.uint32)
        return (uv + jnp.uint32(0x7FFF) + ((uv >> 16) & jnp.uint32(1))) >> 16

    packed = rb(y[:, :dh]) | (rb(y[:, dh:]) << 16)
    return jax.lax.bitcast_convert_type(packed, jnp.int32)


def _router_body(x_ref, gwp_ref, logits_ref, meta_ref, xp_ref):
    x = x_ref[...]
    xp_ref[...] = _pack_bf16(x)
    logits = jax.lax.dot_general(
        x, gwp_ref[...], (((1,), (1,)), ((), ())),
        preferred_element_type=jnp.float32)
    col = jax.lax.broadcasted_iota(jnp.int32, logits.shape, 1)
    valid = col < NUM_EXPERTS
    l = jnp.where(valid, logits, _NEG)
    m1 = jnp.max(l, axis=1, keepdims=True)
    idx1 = jnp.min(jnp.where(l == m1, col, 128), axis=1, keepdims=True)
    l2 = jnp.where(col == idx1, _NEG, l)
    m2 = jnp.max(l2, axis=1, keepdims=True)
    idx2 = jnp.min(jnp.where(l2 == m2, col, 128), axis=1, keepdims=True)
    b = jnp.exp(m2 - m1)
    w1 = 1.0 / (1.0 + b)
    w2 = b * w1
    logits_ref[...] = logits
    meta_ref[...] = jnp.where(col == 0, idx1.astype(jnp.float32),
                    jnp.where(col == 1, idx2.astype(jnp.float32),
                    jnp.where(col == 2, w1,
                    jnp.where(col == 3, w2, 0.0))))


def _router(x, gate_w):
    T, D = x.shape
    TN = 256
    gwp = jnp.zeros((128, D), jnp.float32).at[:NUM_EXPERTS].set(gate_w)
    return pl.pallas_call(
        _router_body,
        grid=(T // TN,),
        in_specs=[
            pl.BlockSpec((TN, D), lambda t: (t, 0)),
            pl.BlockSpec((128, D), lambda t: (0, 0)),
        ],
        out_specs=(pl.BlockSpec((TN, 128), lambda t: (t, 0)),
                   pl.BlockSpec((TN, 128), lambda t: (t, 0)),
                   pl.BlockSpec((TN, D // 2), lambda t: (t, 0))),
        out_shape=(jax.ShapeDtypeStruct((T, 128), jnp.float32),
                   jax.ShapeDtypeStruct((T, 128), jnp.float32),
                   jax.ShapeDtypeStruct((T, D // 2), jnp.int32)),
    )(x, gwp)


def _meta_body(ep_ref, pos_ref, steps_ref, *, tm):
    ep = ep_ref[...]
    R = ep.shape[0]
    i128 = jax.lax.broadcasted_iota(jnp.int32, (128, 128), 0)
    j128 = jax.lax.broadcasted_iota(jnp.int32, (128, 128), 1)
    ut = (i128 <= j128).astype(jnp.float32)
    iR = jax.lax.broadcasted_iota(jnp.int32, (R, R), 0)
    jR = jax.lax.broadcasted_iota(jnp.int32, (R, R), 1)
    slt = (jR < iR).astype(jnp.float32)

    pos = jnp.zeros(ep.shape, jnp.float32)
    tile_off = jnp.float32(0.0)
    tile_offs = []
    for e in range(NUM_EXPERTS):
        m = (ep == e).astype(jnp.float32)
        pin = jnp.dot(m, ut, preferred_element_type=jnp.float32)
        s_col = pin[:, 127:128]
        carry = jnp.dot(slt, jnp.broadcast_to(s_col, ep.shape),
                        preferred_element_type=jnp.float32)
        rank = pin - m + carry
        cnt = jnp.sum(m)
        pos = pos + m * (tile_off * tm + rank)
        tile_offs.append(tile_off)
        tile_off = tile_off + jnp.ceil(cnt / tm)
    total_tiles = tile_off

    ivec = jax.lax.broadcasted_iota(jnp.int32, (1, 128), 1).astype(jnp.float32)
    icl = jnp.minimum(ivec, total_tiles - 1.0)
    sexp = -jnp.ones((1, 128), jnp.float32)
    for e in range(NUM_EXPERTS):
        sexp = sexp + (icl >= tile_offs[e]).astype(jnp.float32)
    svalid = (ivec < total_tiles).astype(jnp.float32)

    pos_ref[...] = pos.astype(jnp.int32)
    r8 = jax.lax.broadcasted_iota(jnp.int32, (8, 128), 0)
    steps = jnp.where(r8 == 0, jnp.broadcast_to(sexp, (8, 128)),
            jnp.where(r8 == 1, jnp.broadcast_to(svalid, (8, 128)), 0.0))
    steps_ref[...] = steps.astype(jnp.int32)


def _metadata(ep, tm):
    R = ep.shape[0]
    return pl.pallas_call(
        functools.partial(_meta_body, tm=tm),
        out_shape=(jax.ShapeDtypeStruct((R, 128), jnp.int32),
                   jax.ShapeDtypeStruct((8, 128), jnp.int32)),
    )(ep)


def _sc_dispatch(x, tid3, pos3, s_pad):
    T, D = x.shape
    nw = tid3.shape[0]
    nchunk, cb = tid3.shape[1], tid3.shape[2]
    info = plsc.get_sparse_core_info()
    nc = info.num_cores

    @functools.partial(
        pl.kernel,
        mesh=plsc.VectorSubcoreMesh(core_axis_name="c", subcore_axis_name="s"),
        out_type=jax.ShapeDtypeStruct((s_pad, D), jnp.int32),
        scratch_types=[
            pltpu.VMEM((nchunk, cb), jnp.int32),
            pltpu.VMEM((nchunk, cb), jnp.int32),
            pltpu.VMEM((cb, D), jnp.int32),
            pltpu.VMEM((cb, D), jnp.int32),
            pltpu.SemaphoreType.DMA,
            pltpu.SemaphoreType.DMA,
            pltpu.SemaphoreType.DMA,
        ],
    )
    def dispatch(x_hbm, tid_hbm, pos_hbm, xs_hbm, idx_t, idx_p,
                 rows0, rows1, semg, sems0, sems1):
        wid = lax.axis_index("s") * nc + lax.axis_index("c")
        pltpu.sync_copy(tid_hbm.at[wid], idx_t)
        pltpu.sync_copy(pos_hbm.at[wid], idx_p)
        bufs = (rows0, rows1)
        ssems = (sems0, sems1)
        hg = {0: pltpu.async_copy(x_hbm.at[idx_t.at[0]], rows0, semg)}
        hs = {}
        for j in range(nchunk):
            buf = bufs[j % 2]
            hg[j].wait()
            hs[j] = pltpu.async_copy(buf, xs_hbm.at[idx_p.at[j]], ssems[j % 2])
            if j + 1 < nchunk:
                if j - 1 >= 0:
                    hs[j - 1].wait()
                hg[j + 1] = pltpu.async_copy(
                    x_hbm.at[idx_t.at[j + 1]], bufs[(j + 1) % 2], semg)
        if nchunk >= 2:
            hs[nchunk - 2].wait()
        hs[nchunk - 1].wait()

    return dispatch(x, tid3, pos3)


def _gmm_body(expert_s, valid_s, xs_ref, wg_ref, wu_ref, wd_ref, ys_ref):
    i = pl.program_id(0)

    @pl.when(valid_s[i] == 1)
    def _compute():
        xb = _unpack_bf16(xs_ref[...]).astype(jnp.bfloat16)
        wg = wg_ref[0].astype(jnp.bfloat16)
        wu = wu_ref[0].astype(jnp.bfloat16)
        wd = wd_ref[0].astype(jnp.bfloat16)
        g = jnp.dot(xb, wg, preferred_element_type=jnp.float32)
        u = jnp.dot(xb, wu, preferred_element_type=jnp.float32)
        h = (g * jax.lax.logistic(g) * u).astype(jnp.bfloat16)
        y = jnp.dot(h, wd, preferred_element_type=jnp.float32)
        dh = y.shape[1] // 2

        def rb(v):
            uv = jax.lax.bitcast_convert_type(v, jnp.uint32)
            return (uv + jnp.uint32(0x7FFF) + ((uv >> 16) & jnp.uint32(1))) >> 16

        packed = rb(y[:, :dh]) | (rb(y[:, dh:]) << 16)
        ys_ref[...] = jax.lax.bitcast_convert_type(packed, jnp.int32)


def _grouped_mlp(xs, w_gate, w_up, w_down, expert_arr, valid_arr, tm):
    s_pad, DP = xs.shape
    D = DP * 2
    E, _, F = w_gate.shape
    G = s_pad // tm
    grid_spec = pltpu.PrefetchScalarGridSpec(
        num_scalar_prefetch=2,
        grid=(G,),
        in_specs=[
            pl.BlockSpec((tm, DP), lambda i, es, vs: (i, 0)),
            pl.BlockSpec((1, D, F), lambda i, es, vs: (es[i], 0, 0)),
            pl.BlockSpec((1, D, F), lambda i, es, vs: (es[i], 0, 0)),
            pl.BlockSpec((1, F, D), lambda i, es, vs: (es[i], 0, 0)),
        ],
        out_specs=pl.BlockSpec((tm, D // 2), lambda i, es, vs: (i, 0)),
    )
    return pl.pallas_call(
        _gmm_body,
        grid_spec=grid_spec,
        out_shape=jax.ShapeDtypeStruct((s_pad, D // 2), jnp.int32),
        compiler_params=pltpu.CompilerParams(
            dimension_semantics=("arbitrary",),
            vmem_limit_bytes=100 * 1024 * 1024),
    )(expert_arr, valid_arr, xs, w_gate, w_up, w_down)


def _sc_collect(ys, p03, p13, T):
    s_pad, D = ys.shape
    nchunk, cb = p03.shape[1], p03.shape[2]
    per_w = nchunk * cb
    info = plsc.get_sparse_core_info()
    nc = info.num_cores

    @functools.partial(
        pl.kernel,
        mesh=plsc.VectorSubcoreMesh(core_axis_name="c", subcore_axis_name="s"),
        out_type=(jax.ShapeDtypeStruct((T, D), jnp.int32),
                  jax.ShapeDtypeStruct((T, D), jnp.int32)),
        scratch_types=[
            pltpu.VMEM((nchunk, cb), jnp.int32),
            pltpu.VMEM((nchunk, cb), jnp.int32),
            pltpu.VMEM((cb, D), jnp.int32),
            pltpu.VMEM((cb, D), jnp.int32),
            pltpu.SemaphoreType.DMA,
            pltpu.SemaphoreType.DMA,
            pltpu.SemaphoreType.DMA,
        ],
    )
    def collect(ys_hbm, p0_hbm, p1_hbm, y0_hbm, y1_hbm, idx0, idx1,
                buf0, buf1, semg, semw0, semw1):
        wid = lax.axis_index("s") * nc + lax.axis_index("c")
        base = wid * per_w
        pltpu.sync_copy(p0_hbm.at[wid], idx0)
        pltpu.sync_copy(p1_hbm.at[wid], idx1)
        steps = []
        for j in range(nchunk):
            steps.append((idx0, j, y0_hbm))
            steps.append((idx1, j, y1_hbm))
        ns = len(steps)
        bufs = (buf0, buf1)
        wsems = (semw0, semw1)
        ix0, j0, _ = steps[0]
        hg = {0: pltpu.async_copy(ys_hbm.at[ix0.at[j0]], buf0, semg)}
        hw = {}
        for s in range(ns):
            buf = bufs[s % 2]
            _, j, dst = steps[s]
            hg[s].wait()
            hw[s] = pltpu.async_copy(
                buf, dst.at[pl.ds(base + j * cb, cb)], wsems[s % 2])
            if s + 1 < ns:
                if s - 1 >= 0:
                    hw[s - 1].wait()
                ixn, jn, _ = steps[s + 1]
                hg[s + 1] = pltpu.async_copy(
                    ys_hbm.at[ixn.at[jn]], bufs[(s + 1) % 2], semg)
        if ns >= 2:
            hw[ns - 2].wait()
        hw[ns - 1].wait()

    return collect(ys, p03, p13)


def _unpack_bf16(p):
    u = jax.lax.bitcast_convert_type(p, jnp.uint32)
    lo = jax.lax.bitcast_convert_type(u << 16, jnp.float32)
    hi = jax.lax.bitcast_convert_type(u & jnp.uint32(0xFFFF0000), jnp.float32)
    return jnp.concatenate([lo, hi], axis=1)


def _combine_body(meta_ref, y0_ref, y1_ref, out_ref):
    out_ref[...] = (_unpack_bf16(y0_ref[...]) * meta_ref[:, 2:3]
                    + _unpack_bf16(y1_ref[...]) * meta_ref[:, 3:4])


def _combine(meta, y0, y1):
    T, DP = y0.shape
    TN = 256
    return pl.pallas_call(
        _combine_body,
        grid=(T // TN,),
        in_specs=[
            pl.BlockSpec((TN, 128), lambda t: (t, 0)),
            pl.BlockSpec((TN, DP), lambda t: (t, 0)),
            pl.BlockSpec((TN, DP), lambda t: (t, 0)),
        ],
        out_specs=pl.BlockSpec((TN, DP * 2), lambda t: (t, 0)),
        out_shape=jax.ShapeDtypeStruct((T, DP * 2), jnp.float32),
    )(meta, y0, y1)


def kernel(hidden_states, gate_w, w_gate, w_up, w_down):
    B, S, D = hidden_states.shape
    T = B * S
    E, _, F = w_gate.shape
    x = hidden_states.reshape(T, D)

    logits128, meta, xp = _router(x, gate_w)
    router_logits = logits128[:, :NUM_EXPERTS]

    idx1 = meta[:, 0].astype(jnp.int32)
    idx2 = meta[:, 1].astype(jnp.int32)
    ep = jnp.concatenate([idx1, idx2]).reshape(TOP_K * T // 128, 128)

    G = (TOP_K * T) // _TM + NUM_EXPERTS
    s_pad = G * _TM
    pos, steps = _metadata(ep, _TM)
    pos_flat = pos.reshape(TOP_K * T)
    expert_arr = steps[0, :G]
    valid_arr = steps[1, :G]

    nw = 32
    tid3 = jnp.tile(jnp.arange(T, dtype=jnp.int32), TOP_K).reshape(nw, -1, 16)
    pos3 = pos_flat.reshape(nw, -1, 16)
    xs = _sc_dispatch(xp, tid3, pos3, s_pad)

    ys = _grouped_mlp(xs, w_gate, w_up, w_down, expert_arr, valid_arr, _TM)

    p03 = pos_flat[:T].reshape(nw, -1, 16)
    p13 = pos_flat[T:].reshape(nw, -1, 16)
    y0, y1 = _sc_collect(ys, p03, p13, T)

    final = _combine(meta, y0, y1)
    return final.reshape(B, S, D), router_logits


# speedup vs baseline: 1.0637x; 1.0356x over previous
import functools

import jax
import jax.numpy as jnp
from jax import lax
from jax.experimental import pallas as pl
from jax.experimental.pallas import tpu as pltpu
from jax.experimental.pallas import tpu_sc as plsc

NUM_EXPERTS = 8
TOP_K = 2
_NEG = -1e30
_TM = 512
_FB = 512


def _pack_bf16(y):
    dh = y.shape[1] // 2

    def rb(v):
        uv = jax.lax.bitcast_convert_type(v, jnp.uint32)
        return (uv + jnp.uint32(0x7FFF) + ((uv >> 16) & jnp.uint32(1))) >> 16

    packed = rb(y[:, :dh]) | (rb(y[:, dh:]) << 16)
    return jax.lax.bitcast_convert_type(packed, jnp.int32)


def _router_body(x_ref, gwp_ref, logits_ref, meta_ref, xp_ref):
    x = x_ref[...]
    xp_ref[...] = _pack_bf16(x)
    logits = jax.lax.dot_general(
        x, gwp_ref[...], (((1,), (1,)), ((), ())),
        preferred_element_type=jnp.float32)
    col = jax.lax.broadcasted_iota(jnp.int32, logits.shape, 1)
    valid = col < NUM_EXPERTS
    l = jnp.where(valid, logits, _NEG)
    m1 = jnp.max(l, axis=1, keepdims=True)
    idx1 = jnp.min(jnp.where(l == m1, col, 128), axis=1, keepdims=True)
    l2 = jnp.where(col == idx1, _NEG, l)
    m2 = jnp.max(l2, axis=1, keepdims=True)
    idx2 = jnp.min(jnp.where(l2 == m2, col, 128), axis=1, keepdims=True)
    b = jnp.exp(m2 - m1)
    w1 = 1.0 / (1.0 + b)
    w2 = b * w1
    logits_ref[...] = logits
    meta_ref[...] = jnp.where(col == 0, idx1.astype(jnp.float32),
                    jnp.where(col == 1, idx2.astype(jnp.float32),
                    jnp.where(col == 2, w1,
                    jnp.where(col == 3, w2, 0.0))))


def _router(x, gate_w):
    T, D = x.shape
    TN = 256
    gwp = jnp.zeros((128, D), jnp.float32).at[:NUM_EXPERTS].set(gate_w)
    return pl.pallas_call(
        _router_body,
        grid=(T // TN,),
        in_specs=[
            pl.BlockSpec((TN, D), lambda t: (t, 0)),
            pl.BlockSpec((128, D), lambda t: (0, 0)),
        ],
        out_specs=(pl.BlockSpec((TN, 128), lambda t: (t, 0)),
                   pl.BlockSpec((TN, 128), lambda t: (t, 0)),
                   pl.BlockSpec((TN, D // 2), lambda t: (t, 0))),
        out_shape=(jax.ShapeDtypeStruct((T, 128), jnp.float32),
                   jax.ShapeDtypeStruct((T, 128), jnp.float32),
                   jax.ShapeDtypeStruct((T, D // 2), jnp.int32)),
    )(x, gwp)


def _meta_body(ep_ref, pos_ref, steps_ref, *, tm):
    ep = ep_ref[...]
    R = ep.shape[0]
    i128 = jax.lax.broadcasted_iota(jnp.int32, (128, 128), 0)
    j128 = jax.lax.broadcasted_iota(jnp.int32, (128, 128), 1)
    ut = (i128 <= j128).astype(jnp.float32)
    iR = jax.lax.broadcasted_iota(jnp.int32, (R, R), 0)
    jR = jax.lax.broadcasted_iota(jnp.int32, (R, R), 1)
    slt = (jR < iR).astype(jnp.float32)

    pos = jnp.zeros(ep.shape, jnp.float32)
    tile_off = jnp.float32(0.0)
    tile_offs = []
    for e in range(NUM_EXPERTS):
        m = (ep == e).astype(jnp.float32)
        pin = jnp.dot(m, ut, preferred_element_type=jnp.float32)
        s_col = pin[:, 127:128]
        carry = jnp.dot(slt, jnp.broadcast_to(s_col, ep.shape),
                        preferred_element_type=jnp.float32)
        rank = pin - m + carry
        cnt = jnp.sum(m)
        pos = pos + m * (tile_off * tm + rank)
        tile_offs.append(tile_off)
        tile_off = tile_off + jnp.ceil(cnt / tm)
    total_tiles = tile_off

    ivec = jax.lax.broadcasted_iota(jnp.int32, (1, 128), 1).astype(jnp.float32)
    icl = jnp.minimum(ivec, total_tiles - 1.0)
    sexp = -jnp.ones((1, 128), jnp.float32)
    for e in range(NUM_EXPERTS):
        sexp = sexp + (icl >= tile_offs[e]).astype(jnp.float32)
    svalid = (ivec < total_tiles).astype(jnp.float32)

    icl_m = jnp.minimum(ivec - 1.0, total_tiles - 1.0)
    eprev = -jnp.ones((1, 128), jnp.float32)
    for e in range(NUM_EXPERTS):
        eprev = eprev + (icl_m >= tile_offs[e]).astype(jnp.float32)
    chg = ((eprev != sexp) & (ivec < total_tiles)).astype(jnp.float32)
    cum = jnp.dot(chg, ut, preferred_element_type=jnp.float32)
    par = cum - 2.0 * jnp.floor(cum * 0.5)
    nxs = jnp.full((1, 128), 999.0, jnp.float32)
    for e in range(NUM_EXPERTS):
        nxs = jnp.minimum(nxs, jnp.where(tile_offs[e] > icl, tile_offs[e], 999.0))
    nxv = (nxs < total_tiles).astype(jnp.float32)
    nxcl = jnp.minimum(nxs, total_tiles - 1.0)
    nx = -jnp.ones((1, 128), jnp.float32)
    for e in range(NUM_EXPERTS):
        nx = nx + (nxcl >= tile_offs[e]).astype(jnp.float32)

    pos_ref[...] = pos.astype(jnp.int32)
    r8 = jax.lax.broadcasted_iota(jnp.int32, (8, 128), 0)
    rows = [sexp, svalid, chg, par, nx, nxv]
    steps = jnp.zeros((8, 128), jnp.float32)
    for k, row in enumerate(rows):
        steps = jnp.where(r8 == k, jnp.broadcast_to(row, (8, 128)), steps)
    steps_ref[...] = steps.astype(jnp.int32)


def _metadata(ep, tm):
    R = ep.shape[0]
    return pl.pallas_call(
        functools.partial(_meta_body, tm=tm),
        out_shape=(jax.ShapeDtypeStruct((R, 128), jnp.int32),
                   jax.ShapeDtypeStruct((8, 128), jnp.int32)),
    )(ep)


def _sc_dispatch(x, tid3, pos3, s_pad):
    T, D = x.shape
    nw = tid3.shape[0]
    nchunk, cb = tid3.shape[1], tid3.shape[2]
    info = plsc.get_sparse_core_info()
    nc = info.num_cores

    @functools.partial(
        pl.kernel,
        mesh=plsc.VectorSubcoreMesh(core_axis_name="c", subcore_axis_name="s"),
        out_type=jax.ShapeDtypeStruct((s_pad, D), jnp.int32),
        scratch_types=[
            pltpu.VMEM((nchunk, cb), jnp.int32),
            pltpu.VMEM((nchunk, cb), jnp.int32),
            pltpu.VMEM((cb, D), jnp.int32),
            pltpu.VMEM((cb, D), jnp.int32),
            pltpu.SemaphoreType.DMA,
            pltpu.SemaphoreType.DMA,
            pltpu.SemaphoreType.DMA,
        ],
    )
    def dispatch(x_hbm, tid_hbm, pos_hbm, xs_hbm, idx_t, idx_p,
                 rows0, rows1, semg, sems0, sems1):
        wid = lax.axis_index("s") * nc + lax.axis_index("c")
        pltpu.sync_copy(tid_hbm.at[wid], idx_t)
        pltpu.sync_copy(pos_hbm.at[wid], idx_p)
        bufs = (rows0, rows1)
        ssems = (sems0, sems1)
        hg = {0: pltpu.async_copy(x_hbm.at[idx_t.at[0]], rows0, semg)}
        hs = {}
        for j in range(nchunk):
            buf = bufs[j % 2]
            hg[j].wait()
            hs[j] = pltpu.async_copy(buf, xs_hbm.at[idx_p.at[j]], ssems[j % 2])
            if j + 1 < nchunk:
                if j - 1 >= 0:
                    hs[j - 1].wait()
                hg[j + 1] = pltpu.async_copy(
                    x_hbm.at[idx_t.at[j + 1]], bufs[(j + 1) % 2], semg)
        if nchunk >= 2:
            hs[nchunk - 2].wait()
        hs[nchunk - 1].wait()

    return dispatch(x, tid3, pos3)


def _gmm_body(es, vs, cg, pr, nx, nv, xs_ref, wg_hbm, wu_hbm, wd_hbm, ys_ref,
              wgb, wub, wdb, sg, su, sd):
    i = pl.program_id(0)
    p = pr[i]

    def _start(e, slot):
        pltpu.make_async_copy(wg_hbm.at[e], wgb.at[slot], sg.at[slot]).start()
        pltpu.make_async_copy(wu_hbm.at[e], wub.at[slot], su.at[slot]).start()
        pltpu.make_async_copy(wd_hbm.at[e], wdb.at[slot], sd.at[slot]).start()

    def _wait(slot):
        pltpu.make_async_copy(wg_hbm.at[0], wgb.at[slot], sg.at[slot]).wait()
        pltpu.make_async_copy(wu_hbm.at[0], wub.at[slot], su.at[slot]).wait()
        pltpu.make_async_copy(wd_hbm.at[0], wdb.at[slot], sd.at[slot]).wait()

    @pl.when((i == 0) & (vs[i] == 1))
    def _first():
        _start(es[i], p)

    @pl.when((cg[i] == 1) & (vs[i] == 1))
    def _boundary():
        _wait(p)

        @pl.when(nv[i] == 1)
        def _prefetch_next():
            _start(nx[i], 1 - p)

    @pl.when(vs[i] == 1)
    def _compute():
        xb = _unpack_bf16(xs_ref[...]).astype(jnp.bfloat16)
        wg = wgb[p].astype(jnp.bfloat16)
        wu = wub[p].astype(jnp.bfloat16)
        wd = wdb[p].astype(jnp.bfloat16)
        g = jnp.dot(xb, wg, preferred_element_type=jnp.float32)
        u = jnp.dot(xb, wu, preferred_element_type=jnp.float32)
        h = (g * jax.lax.logistic(g) * u).astype(jnp.bfloat16)
        y = jnp.dot(h, wd, preferred_element_type=jnp.float32)
        dh = y.shape[1] // 2

        def rb(v):
            uv = jax.lax.bitcast_convert_type(v, jnp.uint32)
            return (uv + jnp.uint32(0x7FFF) + ((uv >> 16) & jnp.uint32(1))) >> 16

        packed = rb(y[:, :dh]) | (rb(y[:, dh:]) << 16)
        ys_ref[...] = jax.lax.bitcast_convert_type(packed, jnp.int32)


def _grouped_mlp(xs, w_gate, w_up, w_down, steps_arrs, tm):
    s_pad, DP = xs.shape
    D = DP * 2
    E, _, F = w_gate.shape
    G = s_pad // tm
    grid_spec = pltpu.PrefetchScalarGridSpec(
        num_scalar_prefetch=6,
        grid=(G,),
        in_specs=[
            pl.BlockSpec((tm, DP), lambda i, *_: (i, 0)),
            pl.BlockSpec(memory_space=pltpu.MemorySpace.HBM),
            pl.BlockSpec(memory_space=pltpu.MemorySpace.HBM),
            pl.BlockSpec(memory_space=pltpu.MemorySpace.HBM),
        ],
        out_specs=pl.BlockSpec((tm, DP), lambda i, *_: (i, 0)),
        scratch_shapes=[
            pltpu.VMEM((2, D, F), jnp.float32),
            pltpu.VMEM((2, D, F), jnp.float32),
            pltpu.VMEM((2, F, D), jnp.float32),
            pltpu.SemaphoreType.DMA((2,)),
            pltpu.SemaphoreType.DMA((2,)),
            pltpu.SemaphoreType.DMA((2,)),
        ],
    )
    return pl.pallas_call(
        _gmm_body,
        grid_spec=grid_spec,
        out_shape=jax.ShapeDtypeStruct((s_pad, DP), jnp.int32),
        compiler_params=pltpu.CompilerParams(
            dimension_semantics=("arbitrary",),
            vmem_limit_bytes=100 * 1024 * 1024),
    )(*steps_arrs, xs, w_gate, w_up, w_down)


def _sc_collect(ys, p03, p13, T):
    s_pad, D = ys.shape
    nchunk, cb = p03.shape[1], p03.shape[2]
    per_w = nchunk * cb
    info = plsc.get_sparse_core_info()
    nc = info.num_cores

    @functools.partial(
        pl.kernel,
        mesh=plsc.VectorSubcoreMesh(core_axis_name="c", subcore_axis_name="s"),
        out_type=(jax.ShapeDtypeStruct((T, D), jnp.int32),
                  jax.ShapeDtypeStruct((T, D), jnp.int32)),
        scratch_types=[
            pltpu.VMEM((nchunk, cb), jnp.int32),
            pltpu.VMEM((nchunk, cb), jnp.int32),
            pltpu.VMEM((cb, D), jnp.int32),
            pltpu.VMEM((cb, D), jnp.int32),
            pltpu.SemaphoreType.DMA,
            pltpu.SemaphoreType.DMA,
            pltpu.SemaphoreType.DMA,
        ],
    )
    def collect(ys_hbm, p0_hbm, p1_hbm, y0_hbm, y1_hbm, idx0, idx1,
                buf0, buf1, semg, semw0, semw1):
        wid = lax.axis_index("s") * nc + lax.axis_index("c")
        base = wid * per_w
        pltpu.sync_copy(p0_hbm.at[wid], idx0)
        pltpu.sync_copy(p1_hbm.at[wid], idx1)
        steps = []
        for j in range(nchunk):
            steps.append((idx0, j, y0_hbm))
            steps.append((idx1, j, y1_hbm))
        ns = len(steps)
        bufs = (buf0, buf1)
        wsems = (semw0, semw1)
        ix0, j0, _ = steps[0]
        hg = {0: pltpu.async_copy(ys_hbm.at[ix0.at[j0]], buf0, semg)}
        hw = {}
        for s in range(ns):
            buf = bufs[s % 2]
            _, j, dst = steps[s]
            hg[s].wait()
            hw[s] = pltpu.async_copy(
                buf, dst.at[pl.ds(base + j * cb, cb)], wsems[s % 2])
            if s + 1 < ns:
                if s - 1 >= 0:
                    hw[s - 1].wait()
                ixn, jn, _ = steps[s + 1]
                hg[s + 1] = pltpu.async_copy(
                    ys_hbm.at[ixn.at[jn]], bufs[(s + 1) % 2], semg)
        if ns >= 2:
            hw[ns - 2].wait()
        hw[ns - 1].wait()

    return collect(ys, p03, p13)


def _unpack_bf16(p):
    u = jax.lax.bitcast_convert_type(p, jnp.uint32)
    lo = jax.lax.bitcast_convert_type(u << 16, jnp.float32)
    hi = jax.lax.bitcast_convert_type(u & jnp.uint32(0xFFFF0000), jnp.float32)
    return jnp.concatenate([lo, hi], axis=1)


def _combine_body(meta_ref, y0_ref, y1_ref, out_ref):
    out_ref[...] = (_unpack_bf16(y0_ref[...]) * meta_ref[:, 2:3]
                    + _unpack_bf16(y1_ref[...]) * meta_ref[:, 3:4])


def _combine(meta, y0, y1):
    T, DP = y0.shape
    TN = 256
    return pl.pallas_call(
        _combine_body,
        grid=(T // TN,),
        in_specs=[
            pl.BlockSpec((TN, 128), lambda t: (t, 0)),
            pl.BlockSpec((TN, DP), lambda t: (t, 0)),
            pl.BlockSpec((TN, DP), lambda t: (t, 0)),
        ],
        out_specs=pl.BlockSpec((TN, DP * 2), lambda t: (t, 0)),
        out_shape=jax.ShapeDtypeStruct((T, DP * 2), jnp.float32),
    )(meta, y0, y1)


def kernel(hidden_states, gate_w, w_gate, w_up, w_down):
    B, S, D = hidden_states.shape
    T = B * S
    E, _, F = w_gate.shape
    x = hidden_states.reshape(T, D)

    logits128, meta, xp = _router(x, gate_w)
    router_logits = logits128[:, :NUM_EXPERTS]

    idx1 = meta[:, 0].astype(jnp.int32)
    idx2 = meta[:, 1].astype(jnp.int32)
    ep = jnp.concatenate([idx1, idx2]).reshape(TOP_K * T // 128, 128)

    G = (TOP_K * T) // _TM + NUM_EXPERTS
    s_pad = G * _TM
    pos, steps = _metadata(ep, _TM)
    pos_flat = pos.reshape(TOP_K * T)
    steps_arrs = [steps[k, :G] for k in range(6)]

    nw = 32
    tid3 = jnp.tile(jnp.arange(T, dtype=jnp.int32), TOP_K).reshape(nw, -1, 16)
    pos3 = pos_flat.reshape(nw, -1, 16)
    xs = _sc_dispatch(xp, tid3, pos3, s_pad)

    ys = _grouped_mlp(xs, w_gate, w_up, w_down, steps_arrs, _TM)

    p03 = pos_flat[:T].reshape(nw, -1, 16)
    p13 = pos_flat[T:].reshape(nw, -1, 16)
    y0, y1 = _sc_collect(ys, p03, p13, T)

    final = _combine(meta, y0, y1)
    return final.reshape(B, S, D), router_logits


# speedup vs baseline: 1.1208x; 1.0537x over previous
import functools

import jax
import jax.numpy as jnp
from jax import lax
from jax.experimental import pallas as pl
from jax.experimental.pallas import tpu as pltpu
from jax.experimental.pallas import tpu_sc as plsc

NUM_EXPERTS = 8
TOP_K = 2
_NEG = -1e30
_TM = 256
_FB = 512


def _pack_bf16(y):
    dh = y.shape[1] // 2

    def rb(v):
        uv = jax.lax.bitcast_convert_type(v, jnp.uint32)
        return (uv + jnp.uint32(0x7FFF) + ((uv >> 16) & jnp.uint32(1))) >> 16

    packed = rb(y[:, :dh]) | (rb(y[:, dh:]) << 16)
    return jax.lax.bitcast_convert_type(packed, jnp.int32)


def _router_body(x_ref, gwp_ref, logits_ref, meta_ref, xp_ref):
    x = x_ref[...]
    xp_ref[...] = _pack_bf16(x)
    logits = jax.lax.dot_general(
        x, gwp_ref[...], (((1,), (1,)), ((), ())),
        preferred_element_type=jnp.float32)
    col = jax.lax.broadcasted_iota(jnp.int32, logits.shape, 1)
    valid = col < NUM_EXPERTS
    l = jnp.where(valid, logits, _NEG)
    m1 = jnp.max(l, axis=1, keepdims=True)
    idx1 = jnp.min(jnp.where(l == m1, col, 128), axis=1, keepdims=True)
    l2 = jnp.where(col == idx1, _NEG, l)
    m2 = jnp.max(l2, axis=1, keepdims=True)
    idx2 = jnp.min(jnp.where(l2 == m2, col, 128), axis=1, keepdims=True)
    b = jnp.exp(m2 - m1)
    w1 = 1.0 / (1.0 + b)
    w2 = b * w1
    logits_ref[...] = logits
    meta_ref[...] = jnp.where(col == 0, idx1.astype(jnp.float32),
                    jnp.where(col == 1, idx2.astype(jnp.float32),
                    jnp.where(col == 2, w1,
                    jnp.where(col == 3, w2, 0.0))))


def _router(x, gate_w):
    T, D = x.shape
    TN = 256
    gwp = jnp.zeros((128, D), jnp.float32).at[:NUM_EXPERTS].set(gate_w)
    return pl.pallas_call(
        _router_body,
        grid=(T // TN,),
        in_specs=[
            pl.BlockSpec((TN, D), lambda t: (t, 0)),
            pl.BlockSpec((128, D), lambda t: (0, 0)),
        ],
        out_specs=(pl.BlockSpec((TN, 128), lambda t: (t, 0)),
                   pl.BlockSpec((TN, 128), lambda t: (t, 0)),
                   pl.BlockSpec((TN, D // 2), lambda t: (t, 0))),
        out_shape=(jax.ShapeDtypeStruct((T, 128), jnp.float32),
                   jax.ShapeDtypeStruct((T, 128), jnp.float32),
                   jax.ShapeDtypeStruct((T, D // 2), jnp.int32)),
    )(x, gwp)


def _meta_body(ep_ref, pos_ref, steps_ref, *, tm):
    ep = ep_ref[...]
    R = ep.shape[0]
    i128 = jax.lax.broadcasted_iota(jnp.int32, (128, 128), 0)
    j128 = jax.lax.broadcasted_iota(jnp.int32, (128, 128), 1)
    ut = (i128 <= j128).astype(jnp.float32)
    iR = jax.lax.broadcasted_iota(jnp.int32, (R, R), 0)
    jR = jax.lax.broadcasted_iota(jnp.int32, (R, R), 1)
    slt = (jR < iR).astype(jnp.float32)

    pos = jnp.zeros(ep.shape, jnp.float32)
    tile_off = jnp.float32(0.0)
    tile_offs = []
    for e in range(NUM_EXPERTS):
        m = (ep == e).astype(jnp.float32)
        pin = jnp.dot(m, ut, preferred_element_type=jnp.float32)
        s_col = pin[:, 127:128]
        carry = jnp.dot(slt, jnp.broadcast_to(s_col, ep.shape),
                        preferred_element_type=jnp.float32)
        rank = pin - m + carry
        cnt = jnp.sum(m)
        pos = pos + m * (tile_off * tm + rank)
        tile_offs.append(tile_off)
        tile_off = tile_off + jnp.ceil(cnt / tm)
    total_tiles = tile_off

    ivec = jax.lax.broadcasted_iota(jnp.int32, (1, 128), 1).astype(jnp.float32)
    icl = jnp.minimum(ivec, total_tiles - 1.0)
    sexp = -jnp.ones((1, 128), jnp.float32)
    for e in range(NUM_EXPERTS):
        sexp = sexp + (icl >= tile_offs[e]).astype(jnp.float32)
    svalid = (ivec < total_tiles).astype(jnp.float32)

    icl_m = jnp.minimum(ivec - 1.0, total_tiles - 1.0)
    eprev = -jnp.ones((1, 128), jnp.float32)
    for e in range(NUM_EXPERTS):
        eprev = eprev + (icl_m >= tile_offs[e]).astype(jnp.float32)
    chg = ((eprev != sexp) & (ivec < total_tiles)).astype(jnp.float32)
    cum = jnp.dot(chg, ut, preferred_element_type=jnp.float32)
    par = cum - 2.0 * jnp.floor(cum * 0.5)
    nxs = jnp.full((1, 128), 999.0, jnp.float32)
    for e in range(NUM_EXPERTS):
        nxs = jnp.minimum(nxs, jnp.where(tile_offs[e] > icl, tile_offs[e], 999.0))
    nxv = (nxs < total_tiles).astype(jnp.float32)
    nxcl = jnp.minimum(nxs, total_tiles - 1.0)
    nx = -jnp.ones((1, 128), jnp.float32)
    for e in range(NUM_EXPERTS):
        nx = nx + (nxcl >= tile_offs[e]).astype(jnp.float32)

    pos_ref[...] = pos.astype(jnp.int32)
    r8 = jax.lax.broadcasted_iota(jnp.int32, (8, 128), 0)
    rows = [sexp, svalid, chg, par, nx, nxv]
    steps = jnp.zeros((8, 128), jnp.float32)
    for k, row in enumerate(rows):
        steps = jnp.where(r8 == k, jnp.broadcast_to(row, (8, 128)), steps)
    steps_ref[...] = steps.astype(jnp.int32)


def _metadata(ep, tm):
    R = ep.shape[0]
    return pl.pallas_call(
        functools.partial(_meta_body, tm=tm),
        out_shape=(jax.ShapeDtypeStruct((R, 128), jnp.int32),
                   jax.ShapeDtypeStruct((8, 128), jnp.int32)),
    )(ep)


def _sc_dispatch(x, tid3, pos3, s_pad):
    T, D = x.shape
    nw = tid3.shape[0]
    nchunk, cb = tid3.shape[1], tid3.shape[2]
    info = plsc.get_sparse_core_info()
    nc = info.num_cores

    @functools.partial(
        pl.kernel,
        mesh=plsc.VectorSubcoreMesh(core_axis_name="c", subcore_axis_name="s"),
        out_type=jax.ShapeDtypeStruct((s_pad, D), jnp.int32),
        scratch_types=[
            pltpu.VMEM((nchunk, cb), jnp.int32),
            pltpu.VMEM((nchunk, cb), jnp.int32),
            pltpu.VMEM((cb, D), jnp.int32),
            pltpu.VMEM((cb, D), jnp.int32),
            pltpu.SemaphoreType.DMA,
            pltpu.SemaphoreType.DMA,
            pltpu.SemaphoreType.DMA,
        ],
    )
    def dispatch(x_hbm, tid_hbm, pos_hbm, xs_hbm, idx_t, idx_p,
                 rows0, rows1, semg, sems0, sems1):
        wid = lax.axis_index("s") * nc + lax.axis_index("c")
        pltpu.sync_copy(tid_hbm.at[wid], idx_t)
        pltpu.sync_copy(pos_hbm.at[wid], idx_p)
        bufs = (rows0, rows1)
        ssems = (sems0, sems1)
        hg = {0: pltpu.async_copy(x_hbm.at[idx_t.at[0]], rows0, semg)}
        hs = {}
        for j in range(nchunk):
            buf = bufs[j % 2]
            hg[j].wait()
            hs[j] = pltpu.async_copy(buf, xs_hbm.at[idx_p.at[j]], ssems[j % 2])
            if j + 1 < nchunk:
                if j - 1 >= 0:
                    hs[j - 1].wait()
                hg[j + 1] = pltpu.async_copy(
                    x_hbm.at[idx_t.at[j + 1]], bufs[(j + 1) % 2], semg)
        if nchunk >= 2:
            hs[nchunk - 2].wait()
        hs[nchunk - 1].wait()

    return dispatch(x, tid3, pos3)


def _gmm_body(es, vs, cg, pr, nx, nv, xs_ref, wg_hbm, wu_hbm, wd_hbm, ys_ref,
              wgb, wub, wdb, sg, su, sd):
    i = pl.program_id(0)
    p = pr[i]

    def _start(e, slot):
        pltpu.make_async_copy(wg_hbm.at[e], wgb.at[slot], sg.at[slot]).start()
        pltpu.make_async_copy(wu_hbm.at[e], wub.at[slot], su.at[slot]).start()
        pltpu.make_async_copy(wd_hbm.at[e], wdb.at[slot], sd.at[slot]).start()

    def _wait(slot):
        pltpu.make_async_copy(wg_hbm.at[0], wgb.at[slot], sg.at[slot]).wait()
        pltpu.make_async_copy(wu_hbm.at[0], wub.at[slot], su.at[slot]).wait()
        pltpu.make_async_copy(wd_hbm.at[0], wdb.at[slot], sd.at[slot]).wait()

    @pl.when((i == 0) & (vs[i] == 1))
    def _first():
        _start(es[i], p)

    @pl.when((cg[i] == 1) & (vs[i] == 1))
    def _boundary():
        _wait(p)

        @pl.when(nv[i] == 1)
        def _prefetch_next():
            _start(nx[i], 1 - p)

    @pl.when(vs[i] == 1)
    def _compute():
        xb = _unpack_bf16(xs_ref[...]).astype(jnp.bfloat16)
        wg = wgb[p].astype(jnp.bfloat16)
        wu = wub[p].astype(jnp.bfloat16)
        wd = wdb[p].astype(jnp.bfloat16)
        g = jnp.dot(xb, wg, preferred_element_type=jnp.float32)
        u = jnp.dot(xb, wu, preferred_element_type=jnp.float32)
        h = (g * jax.lax.logistic(g) * u).astype(jnp.bfloat16)
        y = jnp.dot(h, wd, preferred_element_type=jnp.float32)
        dh = y.shape[1] // 2

        def rb(v):
            uv = jax.lax.bitcast_convert_type(v, jnp.uint32)
            return (uv + jnp.uint32(0x7FFF) + ((uv >> 16) & jnp.uint32(1))) >> 16

        packed = rb(y[:, :dh]) | (rb(y[:, dh:]) << 16)
        ys_ref[...] = jax.lax.bitcast_convert_type(packed, jnp.int32)


def _grouped_mlp(xs, w_gate, w_up, w_down, steps_arrs, tm):
    s_pad, DP = xs.shape
    D = DP * 2
    E, _, F = w_gate.shape
    G = s_pad // tm
    grid_spec = pltpu.PrefetchScalarGridSpec(
        num_scalar_prefetch=6,
        grid=(G,),
        in_specs=[
            pl.BlockSpec((tm, DP), lambda i, *_: (i, 0)),
            pl.BlockSpec(memory_space=pltpu.MemorySpace.HBM),
            pl.BlockSpec(memory_space=pltpu.MemorySpace.HBM),
            pl.BlockSpec(memory_space=pltpu.MemorySpace.HBM),
        ],
        out_specs=pl.BlockSpec((tm, DP), lambda i, *_: (i, 0)),
        scratch_shapes=[
            pltpu.VMEM((2, D, F), jnp.float32),
            pltpu.VMEM((2, D, F), jnp.float32),
            pltpu.VMEM((2, F, D), jnp.float32),
            pltpu.SemaphoreType.DMA((2,)),
            pltpu.SemaphoreType.DMA((2,)),
            pltpu.SemaphoreType.DMA((2,)),
        ],
    )
    return pl.pallas_call(
        _gmm_body,
        grid_spec=grid_spec,
        out_shape=jax.ShapeDtypeStruct((s_pad, DP), jnp.int32),
        compiler_params=pltpu.CompilerParams(
            dimension_semantics=("arbitrary",),
            vmem_limit_bytes=100 * 1024 * 1024),
    )(*steps_arrs, xs, w_gate, w_up, w_down)


def _sc_collect(ys, p03, p13, T):
    s_pad, D = ys.shape
    nchunk, cb = p03.shape[1], p03.shape[2]
    per_w = nchunk * cb
    info = plsc.get_sparse_core_info()
    nc = info.num_cores

    @functools.partial(
        pl.kernel,
        mesh=plsc.VectorSubcoreMesh(core_axis_name="c", subcore_axis_name="s"),
        out_type=(jax.ShapeDtypeStruct((T, D), jnp.int32),
                  jax.ShapeDtypeStruct((T, D), jnp.int32)),
        scratch_types=[
            pltpu.VMEM((nchunk, cb), jnp.int32),
            pltpu.VMEM((nchunk, cb), jnp.int32),
            pltpu.VMEM((cb, D), jnp.int32),
            pltpu.VMEM((cb, D), jnp.int32),
            pltpu.SemaphoreType.DMA,
            pltpu.SemaphoreType.DMA,
            pltpu.SemaphoreType.DMA,
        ],
    )
    def collect(ys_hbm, p0_hbm, p1_hbm, y0_hbm, y1_hbm, idx0, idx1,
                buf0, buf1, semg, semw0, semw1):
        wid = lax.axis_index("s") * nc + lax.axis_index("c")
        base = wid * per_w
        pltpu.sync_copy(p0_hbm.at[wid], idx0)
        pltpu.sync_copy(p1_hbm.at[wid], idx1)
        steps = []
        for j in range(nchunk):
            steps.append((idx0, j, y0_hbm))
            steps.append((idx1, j, y1_hbm))
        ns = len(steps)
        bufs = (buf0, buf1)
        wsems = (semw0, semw1)
        ix0, j0, _ = steps[0]
        hg = {0: pltpu.async_copy(ys_hbm.at[ix0.at[j0]], buf0, semg)}
        hw = {}
        for s in range(ns):
            buf = bufs[s % 2]
            _, j, dst = steps[s]
            hg[s].wait()
            hw[s] = pltpu.async_copy(
                buf, dst.at[pl.ds(base + j * cb, cb)], wsems[s % 2])
            if s + 1 < ns:
                if s - 1 >= 0:
                    hw[s - 1].wait()
                ixn, jn, _ = steps[s + 1]
                hg[s + 1] = pltpu.async_copy(
                    ys_hbm.at[ixn.at[jn]], bufs[(s + 1) % 2], semg)
        if ns >= 2:
            hw[ns - 2].wait()
        hw[ns - 1].wait()

    return collect(ys, p03, p13)


def _unpack_bf16(p):
    u = jax.lax.bitcast_convert_type(p, jnp.uint32)
    lo = jax.lax.bitcast_convert_type(u << 16, jnp.float32)
    hi = jax.lax.bitcast_convert_type(u & jnp.uint32(0xFFFF0000), jnp.float32)
    return jnp.concatenate([lo, hi], axis=1)


def _combine_body(meta_ref, y0_ref, y1_ref, out_ref):
    out_ref[...] = (_unpack_bf16(y0_ref[...]) * meta_ref[:, 2:3]
                    + _unpack_bf16(y1_ref[...]) * meta_ref[:, 3:4])


def _combine(meta, y0, y1):
    T, DP = y0.shape
    TN = 256
    return pl.pallas_call(
        _combine_body,
        grid=(T // TN,),
        in_specs=[
            pl.BlockSpec((TN, 128), lambda t: (t, 0)),
            pl.BlockSpec((TN, DP), lambda t: (t, 0)),
            pl.BlockSpec((TN, DP), lambda t: (t, 0)),
        ],
        out_specs=pl.BlockSpec((TN, DP * 2), lambda t: (t, 0)),
        out_shape=jax.ShapeDtypeStruct((T, DP * 2), jnp.float32),
    )(meta, y0, y1)


def kernel(hidden_states, gate_w, w_gate, w_up, w_down):
    B, S, D = hidden_states.shape
    T = B * S
    E, _, F = w_gate.shape
    x = hidden_states.reshape(T, D)

    logits128, meta, xp = _router(x, gate_w)
    router_logits = logits128[:, :NUM_EXPERTS]

    idx1 = meta[:, 0].astype(jnp.int32)
    idx2 = meta[:, 1].astype(jnp.int32)
    ep = jnp.concatenate([idx1, idx2]).reshape(TOP_K * T // 128, 128)

    G = (TOP_K * T) // _TM + NUM_EXPERTS
    s_pad = G * _TM
    pos, steps = _metadata(ep, _TM)
    pos_flat = pos.reshape(TOP_K * T)
    steps_arrs = [steps[k, :G] for k in range(6)]

    nw = 32
    tid3 = jnp.tile(jnp.arange(T, dtype=jnp.int32), TOP_K).reshape(nw, -1, 16)
    pos3 = pos_flat.reshape(nw, -1, 16)
    xs = _sc_dispatch(xp, tid3, pos3, s_pad)

    ys = _grouped_mlp(xs, w_gate, w_up, w_down, steps_arrs, _TM)

    p03 = pos_flat[:T].reshape(nw, -1, 16)
    p13 = pos_flat[T:].reshape(nw, -1, 16)
    y0, y1 = _sc_collect(ys, p03, p13, T)

    final = _combine(meta, y0, y1)
    return final.reshape(B, S, D), router_logits
